# Initial kernel scaffold; baseline (speedup 1.0000x reference)
import jax
import jax.numpy as jnp
from jax.experimental import pallas as pl


def kernel(encoded_text, table, W, b, gamma, beta):
    raise NotImplementedError("write your pallas kernel here")


# speedup vs baseline: 2.6928x; 2.6928x over previous
import functools

import jax
import jax.numpy as jnp
from jax import lax
from jax.experimental import pallas as pl
from jax.experimental.pallas import tpu as pltpu
from jax.experimental.pallas import tpu_sc as plsc

VOCAB = 1000000
EMB = 32
OUT = 64
B = 16384
L = 50

NC = 2
NS = 16
NW = NC * NS
RPW = B // NW
CH = 32
NCHUNK = RPW // CH

_MESH = plsc.VectorSubcoreMesh(
    core_axis_name="c", subcore_axis_name="s", num_cores=NC, num_subcores=NS
)


@functools.partial(
    pl.kernel,
    out_type=jax.ShapeDtypeStruct((B, EMB), jnp.float32),
    mesh=_MESH,
    scratch_types=[
        pltpu.VMEM((CH, L), jnp.int32),
        pltpu.VMEM((CH, L, EMB), jnp.float32),
        pltpu.VMEM((CH, EMB), jnp.float32),
        pltpu.SemaphoreType.DMA,
    ],
    compiler_params=pltpu.CompilerParams(use_tc_tiling_on_sc=False),
)
def _sc_sum(idx_hbm, table_hbm, out_hbm, idx_v, rows_v, sum_v, sem):
    wid = lax.axis_index("s") * NC + lax.axis_index("c")
    base = wid * RPW

    def chunk_body(g, carry):
        r0 = base + g * CH
        pltpu.sync_copy(idx_hbm.at[pl.ds(r0, CH)], idx_v)
        descs = [
            pltpu.async_copy(table_hbm.at[idx_v.at[j]], rows_v.at[j], sem)
            for j in range(CH)
        ]
        for dsc in descs:
            dsc.wait()

        def row_body(j, c2):
            a0 = rows_v[j, 0, pl.ds(0, 16)]
            a1 = rows_v[j, 0, pl.ds(16, 16)]
            for t in range(1, L):
                a0 = a0 + rows_v[j, t, pl.ds(0, 16)]
                a1 = a1 + rows_v[j, t, pl.ds(16, 16)]
            sum_v[j, pl.ds(0, 16)] = a0
            sum_v[j, pl.ds(16, 16)] = a1
            return c2

        lax.fori_loop(0, CH, row_body, 0, unroll=False)
        pltpu.sync_copy(sum_v, out_hbm.at[pl.ds(r0, CH)])
        return carry

    lax.fori_loop(0, NCHUNK, chunk_body, 0, unroll=False)


_BLK = 2048


def _tc_body(enc_ref, sums_ref, t0_ref, w_ref, b_ref, g_ref, be_ref, o_ref):
    enc = enc_ref[...]
    nz = jnp.sum((enc != 0).astype(jnp.float32), axis=1, keepdims=True)
    n0 = jnp.float32(L) - nz
    pooled = (sums_ref[...] - n0 * t0_ref[...]) / nz
    proj = jnp.dot(pooled, w_ref[...], preferred_element_type=jnp.float32)
    proj = proj + b_ref[...]
    mean = jnp.mean(proj, axis=-1, keepdims=True)
    var = jnp.mean((proj - mean) ** 2, axis=-1, keepdims=True)
    o_ref[...] = (proj - mean) * lax.rsqrt(var + 1e-3) * g_ref[...] + be_ref[...]


_tc_finish = pl.pallas_call(
    _tc_body,
    grid=(B // _BLK,),
    in_specs=[
        pl.BlockSpec((_BLK, L), lambda i: (i, 0)),
        pl.BlockSpec((_BLK, EMB), lambda i: (i, 0)),
        pl.BlockSpec((1, EMB), lambda i: (0, 0)),
        pl.BlockSpec((EMB, OUT), lambda i: (0, 0)),
        pl.BlockSpec((1, OUT), lambda i: (0, 0)),
        pl.BlockSpec((1, OUT), lambda i: (0, 0)),
        pl.BlockSpec((1, OUT), lambda i: (0, 0)),
    ],
    out_specs=pl.BlockSpec((_BLK, OUT), lambda i: (i, 0)),
    out_shape=jax.ShapeDtypeStruct((B, OUT), jnp.float32),
)


def kernel(encoded_text, table, W, b, gamma, beta):
    idx = encoded_text.astype(jnp.int32)
    sums = _sc_sum(idx, table)
    t0 = table[0:1]
    return _tc_finish(
        idx,
        sums,
        t0,
        W,
        b.reshape(1, OUT),
        gamma.reshape(1, OUT),
        beta.reshape(1, OUT),
    )
